# baseline (speedup 1.0000x reference)
import functools

import jax
import jax.numpy as jnp
from jax import lax
from jax.experimental import pallas as pl
from jax.experimental.pallas import tpu as pltpu

NUM_EXPERTS = 8
HIDDEN = 768
CHUNK = 1024
NBUF = 4
NUM_TC = 2


def _router_body(x_hbm, w_hbm, p_hbm, oh_hbm, w_v, bufs, p_v, oh_v, sems, osem):
    cid = lax.axis_index("core")
    t_core = x_hbm.shape[0] // NUM_TC
    nc = t_core // CHUNK
    base = cid * t_core

    pltpu.make_async_copy(w_hbm, w_v, osem.at[0]).start()
    pltpu.make_async_copy(w_hbm, w_v, osem.at[0]).wait()

    def start(c):
        slot = lax.rem(c, NBUF)
        pltpu.make_async_copy(
            x_hbm.at[pl.ds(base + c * CHUNK, CHUNK), :],
            bufs.at[slot],
            sems.at[slot],
        ).start()

    for k in range(NBUF):
        start(k)

    def step(i, carry):
        slot = lax.rem(i, NBUF)
        pltpu.make_async_copy(
            x_hbm.at[pl.ds(base + i * CHUNK, CHUNK), :],
            bufs.at[slot],
            sems.at[slot],
        ).wait()
        logits = jnp.dot(bufs[slot], w_v[...], preferred_element_type=jnp.float32)
        m = jnp.max(logits, axis=-1, keepdims=True)
        unn = jnp.exp(logits - m)
        s = jnp.sum(unn, axis=-1, keepdims=True)
        probs = unn / s
        sl = pl.ds(i * CHUNK, CHUNK)
        p_v[sl, :] = jnp.max(probs, axis=-1, keepdims=True)
        idx = jnp.argmax(probs, axis=-1)
        iota = lax.broadcasted_iota(jnp.int32, probs.shape, 1)
        oh_v[sl, :] = (iota == idx[:, None]).astype(jnp.int32)

        @pl.when(i + NBUF < nc)
        def _next():
            start(i + NBUF)

        return carry

    lax.fori_loop(0, nc, step, 0)

    pltpu.make_async_copy(p_v, p_hbm.at[pl.ds(base, t_core), :], osem.at[0]).start()
    pltpu.make_async_copy(oh_v, oh_hbm.at[pl.ds(base, t_core), :], osem.at[1]).start()
    pltpu.make_async_copy(p_v, p_hbm.at[pl.ds(base, t_core), :], osem.at[0]).wait()
    pltpu.make_async_copy(oh_v, oh_hbm.at[pl.ds(base, t_core), :], osem.at[1]).wait()


def kernel(hidden_states, W):
    B, S, H = hidden_states.shape
    T = B * S
    t_core = T // NUM_TC
    x = hidden_states.reshape(T, H)
    mesh = pltpu.create_tensorcore_mesh("core", num_cores=NUM_TC)
    probs, onehot = pl.kernel(
        _router_body,
        out_type=[
            jax.ShapeDtypeStruct((T, 1), jnp.float32),
            jax.ShapeDtypeStruct((T, NUM_EXPERTS), jnp.int32),
        ],
        mesh=mesh,
        scratch_types=[
            pltpu.VMEM((H, NUM_EXPERTS), jnp.float32),
            pltpu.VMEM((NBUF, CHUNK, H), jnp.float32),
            pltpu.VMEM((t_core, 1), jnp.float32),
            pltpu.VMEM((t_core, NUM_EXPERTS), jnp.int32),
            pltpu.SemaphoreType.DMA((NBUF,)),
            pltpu.SemaphoreType.DMA((2,)),
        ],
    )(x, W)
    p_out = probs.reshape(B, S, 1)
    oh_out = onehot.reshape(B, S, 1, NUM_EXPERTS).astype(jnp.int64)
    return (p_out, oh_out, p_out)


# speedup vs baseline: 1.0330x; 1.0330x over previous
import jax
import jax.numpy as jnp
from jax import lax
from jax.experimental import pallas as pl
from jax.experimental.pallas import tpu as pltpu

NUM_EXPERTS = 8
HIDDEN = 768
CHUNK = 1024
NBUF = 4


def _router_body(x_hbm, w_ref, p_ref, oh_ref, bufs, sems):
    r = pl.program_id(0)
    nc = pl.num_programs(0) * NBUF

    def start(c, k):
        pltpu.make_async_copy(
            x_hbm.at[pl.ds(c * CHUNK, CHUNK), :],
            bufs.at[k],
            sems.at[k],
        ).start(priority=k % 2)

    @pl.when(r == 0)
    def _prime():
        for k in range(NBUF):
            start(k, k)

    for k in range(NBUF):
        c = r * NBUF + k
        pltpu.make_async_copy(
            x_hbm.at[pl.ds(c * CHUNK, CHUNK), :], bufs.at[k], sems.at[k]
        ).wait()
        logits = jnp.dot(bufs[k], w_ref[...], preferred_element_type=jnp.float32)
        m = jnp.max(logits, axis=-1, keepdims=True)
        unn = jnp.exp(logits - m)
        s = jnp.sum(unn, axis=-1, keepdims=True)
        probs = unn / s
        sl = pl.ds(k * CHUNK, CHUNK)
        p_ref[sl, :] = jnp.max(probs, axis=-1, keepdims=True)
        idx = jnp.argmax(probs, axis=-1)
        iota = lax.broadcasted_iota(jnp.int32, probs.shape, 1)
        oh_ref[sl, :] = (iota == idx[:, None]).astype(jnp.int32)

        @pl.when(c + NBUF < nc)
        def _next():
            start(c + NBUF, k)


def kernel(hidden_states, W):
    B, S, H = hidden_states.shape
    T = B * S
    x = hidden_states.reshape(T, H)
    grid = (T // (CHUNK * NBUF),)
    probs, onehot = pl.pallas_call(
        _router_body,
        grid=grid,
        in_specs=[
            pl.BlockSpec(memory_space=pltpu.MemorySpace.HBM),
            pl.BlockSpec((HIDDEN, NUM_EXPERTS), lambda i: (0, 0)),
        ],
        out_specs=[
            pl.BlockSpec((CHUNK * NBUF, 1), lambda i: (i, 0)),
            pl.BlockSpec((CHUNK * NBUF, NUM_EXPERTS), lambda i: (i, 0)),
        ],
        out_shape=[
            jax.ShapeDtypeStruct((T, 1), jnp.float32),
            jax.ShapeDtypeStruct((T, NUM_EXPERTS), jnp.int32),
        ],
        scratch_shapes=[
            pltpu.VMEM((NBUF, CHUNK, HIDDEN), jnp.float32),
            pltpu.SemaphoreType.DMA((NBUF,)),
        ],
        compiler_params=pltpu.CompilerParams(
            dimension_semantics=("arbitrary",),
        ),
    )(x, W)
    p_out = probs.reshape(B, S, 1)
    oh_out = onehot.reshape(B, S, 1, NUM_EXPERTS).astype(jnp.int64)
    return (p_out, oh_out, p_out)
